# initial kernel scaffold (unmeasured)
import jax
import jax.numpy as jnp
from jax import lax
from jax.experimental import pallas as pl
from jax.experimental.pallas import tpu as pltpu

N_DEV = 4
N_GLOBAL = 4096
EPS = 1e-5


def kernel(x, gamma):
    m, n_loc = x.shape
    gamma2d = gamma.reshape(1, n_loc)

    def body(x_ref, g_ref, out_ref, p_ref, gather_ref, send_sems, recv_sems):
        my_pos = lax.axis_index("i")

        xs = x_ref[:, :]
        p_ref[:, :] = jnp.sum(xs * xs, axis=1, keepdims=True)

        rdmas = []
        for k in range(1, N_DEV):
            dst = (my_pos + k) % N_DEV
            rdma = pltpu.make_async_remote_copy(
                src_ref=p_ref,
                dst_ref=gather_ref.at[k - 1],
                send_sem=send_sems.at[k - 1],
                recv_sem=recv_sems.at[k - 1],
                device_id=(dst,),
                device_id_type=pl.DeviceIdType.MESH,
            )
            rdma.start()
            rdmas.append(rdma)
        for rdma in rdmas:
            rdma.wait()

        total = p_ref[:, :] + gather_ref[0] + gather_ref[1] + gather_ref[2]
        rms = jnp.sqrt(total * (1.0 / N_GLOBAL) + EPS)
        out_ref[:, :] = g_ref[:, :] * xs / rms

    return pl.pallas_call(
        body,
        out_shape=jax.ShapeDtypeStruct((m, n_loc), jnp.float32),
        in_specs=[
            pl.BlockSpec(memory_space=pltpu.VMEM),
            pl.BlockSpec(memory_space=pltpu.VMEM),
        ],
        out_specs=pl.BlockSpec(memory_space=pltpu.VMEM),
        scratch_shapes=[
            pltpu.VMEM((m, 1), jnp.float32),
            pltpu.VMEM((N_DEV - 1, m, 1), jnp.float32),
            pltpu.SemaphoreType.DMA((N_DEV - 1,)),
            pltpu.SemaphoreType.DMA((N_DEV - 1,)),
        ],
    )(x, gamma2d)


# baseline (device time: 80143 ns/iter reference)
import jax
import jax.numpy as jnp
from jax import lax
from jax.experimental import pallas as pl
from jax.experimental.pallas import tpu as pltpu

N_DEV = 4
N_GLOBAL = 4096
EPS = 1e-5


def kernel(x, gamma):
    m, n_loc = x.shape
    gamma2d = gamma.reshape(1, n_loc)

    def body(x_ref, g_ref, out_ref, p_ref, gather_ref, send_sems, recv_sems):
        my_pos = lax.axis_index("i")

        xs = x_ref[:, :]
        p_ref[:, :] = jnp.sum(xs * xs, axis=1, keepdims=True)

        rdmas = []
        for k in range(1, N_DEV):
            dst = (my_pos + k) % N_DEV
            rdma = pltpu.make_async_remote_copy(
                src_ref=p_ref,
                dst_ref=gather_ref.at[k - 1],
                send_sem=send_sems.at[k - 1],
                recv_sem=recv_sems.at[k - 1],
                device_id=(dst,),
                device_id_type=pl.DeviceIdType.MESH,
            )
            rdma.start()
            rdmas.append(rdma)
        for rdma in rdmas:
            rdma.wait()

        total = p_ref[:, :] + gather_ref[0] + gather_ref[1] + gather_ref[2]
        rms = jnp.sqrt(total * (1.0 / N_GLOBAL) + EPS)
        out_ref[:, :] = g_ref[:, :] * xs / rms

    return pl.pallas_call(
        body,
        out_shape=jax.ShapeDtypeStruct((m, n_loc), jnp.float32),
        in_specs=[
            pl.BlockSpec(memory_space=pltpu.VMEM),
            pl.BlockSpec(memory_space=pltpu.VMEM),
        ],
        out_specs=pl.BlockSpec(memory_space=pltpu.VMEM),
        scratch_shapes=[
            pltpu.VMEM((m, 1), jnp.float32),
            pltpu.VMEM((N_DEV - 1, m, 1), jnp.float32),
            pltpu.SemaphoreType.DMA((N_DEV - 1,)),
            pltpu.SemaphoreType.DMA((N_DEV - 1,)),
        ],
        compiler_params=pltpu.CompilerParams(
            vmem_limit_bytes=60 * 1024 * 1024,
        ),
    )(x, gamma2d)


# device time: 79469 ns/iter; 1.0085x vs baseline; 1.0085x over previous
import jax
import jax.numpy as jnp
from jax import lax
from jax.experimental import pallas as pl
from jax.experimental.pallas import tpu as pltpu

N_DEV = 4
N_GLOBAL = 4096
EPS = 1e-5


def kernel(x, gamma):
    m, n_loc = x.shape
    gamma2d = gamma.reshape(1, n_loc)

    def body(x_ref, g_ref, out_ref, p_ref, gather_ref, send_sems, recv_sems):
        my_pos = lax.axis_index("i")

        xs = x_ref[:, :]
        p_ref[:, :] = jnp.sum(xs * xs, axis=1, keepdims=True)

        rdmas = []
        for k in range(1, N_DEV):
            dst = (my_pos + k) % N_DEV
            rdma = pltpu.make_async_remote_copy(
                src_ref=p_ref,
                dst_ref=gather_ref.at[k - 1],
                send_sem=send_sems.at[k - 1],
                recv_sem=recv_sems.at[k - 1],
                device_id=(dst,),
                device_id_type=pl.DeviceIdType.MESH,
            )
            rdma.start()
            rdmas.append(rdma)

        out_ref[:, :] = g_ref[:, :] * xs

        for rdma in rdmas:
            rdma.wait()

        total = p_ref[:, :] + gather_ref[0] + gather_ref[1] + gather_ref[2]
        inv_rms = lax.rsqrt(total * (1.0 / N_GLOBAL) + EPS)
        out_ref[:, :] = out_ref[:, :] * inv_rms

    return pl.pallas_call(
        body,
        out_shape=jax.ShapeDtypeStruct((m, n_loc), jnp.float32),
        in_specs=[
            pl.BlockSpec(memory_space=pltpu.VMEM),
            pl.BlockSpec(memory_space=pltpu.VMEM),
        ],
        out_specs=pl.BlockSpec(memory_space=pltpu.VMEM),
        scratch_shapes=[
            pltpu.VMEM((m, 1), jnp.float32),
            pltpu.VMEM((N_DEV - 1, m, 1), jnp.float32),
            pltpu.SemaphoreType.DMA((N_DEV - 1,)),
            pltpu.SemaphoreType.DMA((N_DEV - 1,)),
        ],
        compiler_params=pltpu.CompilerParams(
            vmem_limit_bytes=60 * 1024 * 1024,
        ),
    )(x, gamma2d)


# device time: 35020 ns/iter; 2.2885x vs baseline; 2.2692x over previous
import jax
import jax.numpy as jnp
from jax import lax
from jax.experimental import pallas as pl
from jax.experimental.pallas import tpu as pltpu

N_DEV = 4
N_GLOBAL = 4096
EPS = 1e-5


def kernel(x, gamma):
    m, n_loc = x.shape
    gamma2d = gamma.reshape(1, n_loc)

    def body(x_ref, g_ref, out_ref, p_ref, gather_ref, send_sems, recv_sems):
        my_pos = lax.axis_index("i")

        xs = x_ref[:, :]
        p_ref[:, :] = jnp.sum(xs * xs, axis=1).reshape(m // 128, 128)

        rdmas = []
        for k in range(1, N_DEV):
            dst = (my_pos + k) % N_DEV
            rdma = pltpu.make_async_remote_copy(
                src_ref=p_ref,
                dst_ref=gather_ref.at[k - 1],
                send_sem=send_sems.at[k - 1],
                recv_sem=recv_sems.at[k - 1],
                device_id=(dst,),
                device_id_type=pl.DeviceIdType.MESH,
            )
            rdma.start()
            rdmas.append(rdma)

        out_ref[:, :] = g_ref[:, :] * xs

        for rdma in rdmas:
            rdma.wait()

        total = p_ref[:, :] + gather_ref[0] + gather_ref[1] + gather_ref[2]
        inv_rms = lax.rsqrt(total * (1.0 / N_GLOBAL) + EPS)
        big = jnp.broadcast_to(inv_rms[:, None, :], (m // 128, 128, 128))
        big = big.reshape(m, 128)
        lane = lax.broadcasted_iota(jnp.int32, (m, 128), 1)
        row_mod = lax.broadcasted_iota(jnp.int32, (m, 128), 0) % 128
        inv_col = jnp.sum(
            jnp.where(lane == row_mod, big, 0.0), axis=1, keepdims=True
        )
        out_ref[:, :] = out_ref[:, :] * inv_col

    return pl.pallas_call(
        body,
        out_shape=jax.ShapeDtypeStruct((m, n_loc), jnp.float32),
        in_specs=[
            pl.BlockSpec(memory_space=pltpu.VMEM),
            pl.BlockSpec(memory_space=pltpu.VMEM),
        ],
        out_specs=pl.BlockSpec(memory_space=pltpu.VMEM),
        scratch_shapes=[
            pltpu.VMEM((m // 128, 128), jnp.float32),
            pltpu.VMEM((N_DEV - 1, m // 128, 128), jnp.float32),
            pltpu.SemaphoreType.DMA((N_DEV - 1,)),
            pltpu.SemaphoreType.DMA((N_DEV - 1,)),
        ],
        compiler_params=pltpu.CompilerParams(
            vmem_limit_bytes=60 * 1024 * 1024,
        ),
    )(x, gamma2d)


# device time: 34333 ns/iter; 2.3343x vs baseline; 1.0200x over previous
import jax
import jax.numpy as jnp
from jax import lax
from jax.experimental import pallas as pl
from jax.experimental.pallas import tpu as pltpu

N_DEV = 4
N_GLOBAL = 4096
EPS = 1e-5
N_CHUNKS = 4


def kernel(x, gamma):
    m, n_loc = x.shape
    gamma2d = gamma.reshape(1, n_loc)
    m_chunk = m // N_CHUNKS

    def body(x_ref, g_ref, out_ref, o_vmem, p_ref, gather_ref,
             send_sems, recv_sems, store_sems):
        my_pos = lax.axis_index("i")

        xs = x_ref[:, :]
        p_ref[:, :] = jnp.sum(xs * xs, axis=1).reshape(m // 128, 128)

        rdmas = []
        for k in range(1, N_DEV):
            dst = (my_pos + k) % N_DEV
            rdma = pltpu.make_async_remote_copy(
                src_ref=p_ref,
                dst_ref=gather_ref.at[k - 1],
                send_sem=send_sems.at[k - 1],
                recv_sem=recv_sems.at[k - 1],
                device_id=(dst,),
                device_id_type=pl.DeviceIdType.MESH,
            )
            rdma.start()
            rdmas.append(rdma)

        o_vmem[:, :] = g_ref[:, :] * xs

        for rdma in rdmas:
            rdma.wait()

        total = p_ref[:, :] + gather_ref[0] + gather_ref[1] + gather_ref[2]
        inv_rms = lax.rsqrt(total * (1.0 / N_GLOBAL) + EPS)
        big = jnp.broadcast_to(inv_rms[:, None, :], (m // 128, 128, 128))
        big = big.reshape(m, 128)
        lane = lax.broadcasted_iota(jnp.int32, (m, 128), 1)
        row_mod = lax.broadcasted_iota(jnp.int32, (m, 128), 0) % 128
        inv_col = jnp.sum(
            jnp.where(lane == row_mod, big, 0.0), axis=1, keepdims=True
        )

        copies = []
        for c in range(N_CHUNKS):
            sl = pl.ds(c * m_chunk, m_chunk)
            o_vmem[sl, :] = o_vmem[sl, :] * inv_col[c * m_chunk:(c + 1) * m_chunk, :]
            cp = pltpu.make_async_copy(
                o_vmem.at[sl], out_ref.at[sl], store_sems.at[c]
            )
            cp.start()
            copies.append(cp)
        for cp in copies:
            cp.wait()

    return pl.pallas_call(
        body,
        out_shape=jax.ShapeDtypeStruct((m, n_loc), jnp.float32),
        in_specs=[
            pl.BlockSpec(memory_space=pltpu.VMEM),
            pl.BlockSpec(memory_space=pltpu.VMEM),
        ],
        out_specs=pl.BlockSpec(memory_space=pl.ANY),
        scratch_shapes=[
            pltpu.VMEM((m, n_loc), jnp.float32),
            pltpu.VMEM((m // 128, 128), jnp.float32),
            pltpu.VMEM((N_DEV - 1, m // 128, 128), jnp.float32),
            pltpu.SemaphoreType.DMA((N_DEV - 1,)),
            pltpu.SemaphoreType.DMA((N_DEV - 1,)),
            pltpu.SemaphoreType.DMA((N_CHUNKS,)),
        ],
        compiler_params=pltpu.CompilerParams(
            vmem_limit_bytes=60 * 1024 * 1024,
        ),
    )(x, gamma2d)


# device time: 29044 ns/iter; 2.7594x vs baseline; 1.1821x over previous
import jax
import jax.numpy as jnp
from jax import lax
from jax.experimental import pallas as pl
from jax.experimental.pallas import tpu as pltpu

N_DEV = 4
N_GLOBAL = 4096
EPS = 1e-5


def _allreduce_inv_rms(x):
    m, n_loc = x.shape

    def body(x_ref, inv_ref, p_ref, gather_ref, send_sems, recv_sems):
        my_pos = lax.axis_index("i")

        xs = x_ref[:, :]
        p_ref[:, :] = jnp.sum(xs * xs, axis=1).reshape(m // 128, 128)

        rdmas = []
        for k in range(1, N_DEV):
            dst = (my_pos + k) % N_DEV
            rdma = pltpu.make_async_remote_copy(
                src_ref=p_ref,
                dst_ref=gather_ref.at[k - 1],
                send_sem=send_sems.at[k - 1],
                recv_sem=recv_sems.at[k - 1],
                device_id=(dst,),
                device_id_type=pl.DeviceIdType.MESH,
            )
            rdma.start()
            rdmas.append(rdma)
        for rdma in rdmas:
            rdma.wait()

        total = p_ref[:, :] + gather_ref[0] + gather_ref[1] + gather_ref[2]
        inv_ref[:, :] = lax.rsqrt(total * (1.0 / N_GLOBAL) + EPS)

    return pl.pallas_call(
        body,
        out_shape=jax.ShapeDtypeStruct((m // 128, 128), jnp.float32),
        in_specs=[pl.BlockSpec(memory_space=pltpu.VMEM)],
        out_specs=pl.BlockSpec(memory_space=pltpu.VMEM),
        scratch_shapes=[
            pltpu.VMEM((m // 128, 128), jnp.float32),
            pltpu.VMEM((N_DEV - 1, m // 128, 128), jnp.float32),
            pltpu.SemaphoreType.DMA((N_DEV - 1,)),
            pltpu.SemaphoreType.DMA((N_DEV - 1,)),
        ],
        compiler_params=pltpu.CompilerParams(
            vmem_limit_bytes=60 * 1024 * 1024,
        ),
    )(x)


def _normalize(x, gamma2d, inv_packed):
    m, n_loc = x.shape

    def body(x_ref, g_ref, inv_ref, out_ref):
        inv_rms = inv_ref[:, :]
        big = jnp.broadcast_to(inv_rms[:, None, :], (m // 128, 128, 128))
        big = big.reshape(m, 128)
        lane = lax.broadcasted_iota(jnp.int32, (m, 128), 1)
        row_mod = lax.broadcasted_iota(jnp.int32, (m, 128), 0) % 128
        inv_col = jnp.sum(
            jnp.where(lane == row_mod, big, 0.0), axis=1, keepdims=True
        )
        out_ref[:, :] = g_ref[:, :] * x_ref[:, :] * inv_col

    return pl.pallas_call(
        body,
        out_shape=jax.ShapeDtypeStruct((m, n_loc), jnp.float32),
        in_specs=[
            pl.BlockSpec(memory_space=pltpu.VMEM),
            pl.BlockSpec(memory_space=pltpu.VMEM),
            pl.BlockSpec(memory_space=pltpu.VMEM),
        ],
        out_specs=pl.BlockSpec(memory_space=pltpu.VMEM),
        compiler_params=pltpu.CompilerParams(
            vmem_limit_bytes=60 * 1024 * 1024,
        ),
    )(x, gamma2d, inv_packed)


def kernel(x, gamma):
    m, n_loc = x.shape
    gamma2d = gamma.reshape(1, n_loc)
    inv_packed = _allreduce_inv_rms(x)
    return _normalize(x, gamma2d, inv_packed)


# device time: 28776 ns/iter; 2.7851x vs baseline; 1.0093x over previous
import jax
import jax.numpy as jnp
from jax import lax
from jax.experimental import pallas as pl
from jax.experimental.pallas import tpu as pltpu

N_DEV = 4
N_GLOBAL = 4096
EPS = 1e-5
GRID = 4


def _allreduce_inv_rms(x):
    m, n_loc = x.shape
    mc = m // GRID
    pc = mc // 128

    def body(x_ref, inv_ref, p_ref, gather_ref, send_sems, recv_sems):
        c = pl.program_id(0)
        xs = x_ref[:, :]
        p_ref[pl.ds(c * pc, pc), :] = jnp.sum(xs * xs, axis=1).reshape(pc, 128)

        @pl.when(c == GRID - 1)
        def _():
            my_pos = lax.axis_index("i")
            rdmas = []
            for k in range(1, N_DEV):
                dst = (my_pos + k) % N_DEV
                rdma = pltpu.make_async_remote_copy(
                    src_ref=p_ref,
                    dst_ref=gather_ref.at[k - 1],
                    send_sem=send_sems.at[k - 1],
                    recv_sem=recv_sems.at[k - 1],
                    device_id=(dst,),
                    device_id_type=pl.DeviceIdType.MESH,
                )
                rdma.start()
                rdmas.append(rdma)
            for rdma in rdmas:
                rdma.wait()

            total = (p_ref[:, :] + gather_ref[0] + gather_ref[1]
                     + gather_ref[2])
            inv_ref[:, :] = lax.rsqrt(total * (1.0 / N_GLOBAL) + EPS)

    return pl.pallas_call(
        body,
        grid=(GRID,),
        out_shape=jax.ShapeDtypeStruct((m // 128, 128), jnp.float32),
        in_specs=[pl.BlockSpec((mc, n_loc), lambda c: (c, 0))],
        out_specs=pl.BlockSpec((m // 128, 128), lambda c: (0, 0)),
        scratch_shapes=[
            pltpu.VMEM((m // 128, 128), jnp.float32),
            pltpu.VMEM((N_DEV - 1, m // 128, 128), jnp.float32),
            pltpu.SemaphoreType.DMA((N_DEV - 1,)),
            pltpu.SemaphoreType.DMA((N_DEV - 1,)),
        ],
        compiler_params=pltpu.CompilerParams(
            vmem_limit_bytes=60 * 1024 * 1024,
        ),
    )(x)


def _normalize(x, gamma2d, inv_packed):
    m, n_loc = x.shape
    mc = m // GRID
    pc = mc // 128

    def body(x_ref, g_ref, inv_ref, out_ref):
        inv_rms = inv_ref[:, :]
        big = jnp.broadcast_to(inv_rms[:, None, :], (pc, 128, 128))
        big = big.reshape(mc, 128)
        lane = lax.broadcasted_iota(jnp.int32, (mc, 128), 1)
        row_mod = lax.broadcasted_iota(jnp.int32, (mc, 128), 0) % 128
        inv_col = jnp.sum(
            jnp.where(lane == row_mod, big, 0.0), axis=1, keepdims=True
        )
        out_ref[:, :] = g_ref[:, :] * x_ref[:, :] * inv_col

    return pl.pallas_call(
        body,
        grid=(GRID,),
        out_shape=jax.ShapeDtypeStruct((m, n_loc), jnp.float32),
        in_specs=[
            pl.BlockSpec((mc, n_loc), lambda c: (c, 0)),
            pl.BlockSpec((1, n_loc), lambda c: (0, 0)),
            pl.BlockSpec((pc, 128), lambda c: (c, 0)),
        ],
        out_specs=pl.BlockSpec((mc, n_loc), lambda c: (c, 0)),
        compiler_params=pltpu.CompilerParams(
            vmem_limit_bytes=60 * 1024 * 1024,
        ),
    )(x, gamma2d, inv_packed)


def kernel(x, gamma):
    m, n_loc = x.shape
    gamma2d = gamma.reshape(1, n_loc)
    inv_packed = _allreduce_inv_rms(x)
    return _normalize(x, gamma2d, inv_packed)
